# Initial kernel scaffold; baseline (speedup 1.0000x reference)
import jax
import jax.numpy as jnp
from jax.experimental import pallas as pl


def kernel(node_feats, edge_index, edge_attr, msg_w1, msg_b1, msg_w2, msg_b2, upd_w1, upd_b1, upd_w2, upd_b2):
    raise NotImplementedError("write your pallas kernel here")


# speedup vs baseline: 3.9365x; 3.9365x over previous
import functools

import jax
import jax.numpy as jnp
from jax import lax
from jax.experimental import pallas as pl
from jax.experimental.pallas import tpu as pltpu
from jax.experimental.pallas import tpu_sc as plsc

N_NODES = 10000
N_EDGES = 320000
D = 128
D_EDGE = 16

NC = 2
NS = 16
NW = NC * NS
EPT = N_EDGES // NW
CH = 80
NCH = EPT // CH
RSTRIDE = 624
_ROW_PIECES = tuple((i * CH, CH) for i in range(8))

DROWS = 80


def _ab_body(x_ref, w_ref, a_ref, b_ref):
    ab = jnp.dot(x_ref[...], w_ref[...], preferred_element_type=jnp.float32)
    a_ref[...] = ab[:, :D]
    b_ref[...] = ab[:, D:]


def _c_body(ea_ref, w_ref, b_ref, o_ref):
    o_ref[...] = (
        jnp.dot(ea_ref[...], w_ref[...], preferred_element_type=jnp.float32)
        + b_ref[...]
    )


def _post_body(sh_ref, sd_ref, x_ref, w2_ref, b2_ref, u1a_ref, u1b_ref,
               ub1_ref, u2_ref, ub2_ref, o_ref):
    s = sh_ref[0] + sh_ref[1]
    deg = sd_ref[0] + sd_ref[1]
    inv = 1.0 / jnp.maximum(deg, 1.0)
    agg = jnp.dot(s, w2_ref[...], preferred_element_type=jnp.float32) * inv
    agg = agg + jnp.where(deg > 0.0, 1.0, 0.0) * b2_ref[...]
    t = jnp.maximum(
        jnp.dot(x_ref[...], u1a_ref[...], preferred_element_type=jnp.float32)
        + jnp.dot(agg, u1b_ref[...], preferred_element_type=jnp.float32)
        + ub1_ref[...],
        0.0,
    )
    o_ref[...] = (
        jnp.dot(t, u2_ref[...], preferred_element_type=jnp.float32) + ub2_ref[...]
    )


def _edge_pass(a_hbm, b_hbm, c_hbm, src_hbm, dst_hbm, out_hbm, deg_hbm,
               sidx, didx, ridx, abuf, bbuf, cbuf, hist, acc, dacc,
               sem_a, sem_b):
    c = lax.axis_index("c")
    s = lax.axis_index("s")
    wid = c * NS + s

    iota16 = lax.iota(jnp.int32, 16)
    zeros = jnp.zeros((16,), jnp.float32)
    ones = zeros + 1.0

    def zrow(i, carry):
        for j in range(D // 16):
            abuf[i, pl.ds(j * 16, 16)] = zeros
            hist[i, pl.ds(j * 16, 16)] = zeros
        return carry

    lax.fori_loop(0, CH, zrow, 0)
    for k in range(DROWS // 16):
        ridx[pl.ds(k * 16, 16)] = iota16 + (k * 16)

    rbase = s * RSTRIDE
    for off, n in _ROW_PIECES:
        pltpu.sync_copy(abuf.at[pl.ds(0, n)], acc.at[pl.ds(rbase + off, n)])

    @pl.when(s == 0)
    def _():
        pltpu.sync_copy(abuf, dacc)

    plsc.subcore_barrier()

    ebase = wid * EPT

    def chunk(k, carry):
        eo = ebase + k * CH
        pltpu.sync_copy(src_hbm.at[pl.ds(eo, CH)], sidx)
        pltpu.sync_copy(dst_hbm.at[pl.ds(eo, CH)], didx)
        ga = pltpu.async_copy(a_hbm.at[sidx], abuf, sem_a)
        gb = pltpu.async_copy(b_hbm.at[didx], bbuf, sem_b)
        pltpu.sync_copy(c_hbm.at[pl.ds(eo, CH)], cbuf)
        ga.wait()
        gb.wait()

        def row(i, rc):
            for j in range(D // 16):
                sl = pl.ds(j * 16, 16)
                v = abuf[i, sl] + bbuf[i, sl] + cbuf[i, sl]
                cbuf[i, sl] = jnp.maximum(v, 0.0)
            return rc

        lax.fori_loop(0, CH, row, 0)
        pltpu.sync_copy(cbuf, acc.at[didx], add=True)

        for j in range(CH // 16):
            dvec = didx[pl.ds(j * 16, 16)]
            for l in range(16):
                d0 = dvec[l]
                r = lax.shift_right_logical(d0, 7)
                jc = lax.bitwise_and(d0, 127) - lax.bitwise_and(d0, 15)
                lane = lax.bitwise_and(d0, 15)
                sl = pl.ds(jc, 16)
                hist[r, sl] = hist[r, sl] + jnp.where(iota16 == lane, 1.0, 0.0)
        return carry

    lax.fori_loop(0, NCH, chunk, 0)

    pltpu.sync_copy(hist, dacc.at[ridx], add=True)
    plsc.subcore_barrier()

    for off, n in _ROW_PIECES:
        pltpu.sync_copy(acc.at[pl.ds(rbase + off, n)],
                        out_hbm.at[c, pl.ds(rbase + off, n)])

    @pl.when(s == 0)
    def _():
        pltpu.sync_copy(dacc, deg_hbm.at[c])


_EDGE_KERNEL_CACHE = []


def _edge_kernel():
    if not _EDGE_KERNEL_CACHE:
        _EDGE_KERNEL_CACHE.append(functools.partial(
            pl.kernel,
            out_type=[
                jax.ShapeDtypeStruct((NC, N_NODES, D), jnp.float32),
                jax.ShapeDtypeStruct((NC, DROWS, D), jnp.float32),
            ],
            mesh=plsc.VectorSubcoreMesh(core_axis_name="c", subcore_axis_name="s",
                                        num_cores=NC, num_subcores=NS),
            scratch_types=[
                pltpu.VMEM((CH,), jnp.int32),
                pltpu.VMEM((CH,), jnp.int32),
                pltpu.VMEM((DROWS,), jnp.int32),
                pltpu.VMEM((CH, D), jnp.float32),
                pltpu.VMEM((CH, D), jnp.float32),
                pltpu.VMEM((CH, D), jnp.float32),
                pltpu.VMEM((DROWS, D), jnp.float32),
                pltpu.VMEM_SHARED((N_NODES, D), jnp.float32),
                pltpu.VMEM_SHARED((DROWS, D), jnp.float32),
                pltpu.SemaphoreType.DMA,
                pltpu.SemaphoreType.DMA,
            ],
        )(_edge_pass))
    return _EDGE_KERNEL_CACHE[0]


def kernel(node_feats, edge_index, edge_attr, msg_w1, msg_b1, msg_w2, msg_b2,
           upd_w1, upd_b1, upd_w2, upd_b2):
    src = edge_index[0].astype(jnp.int32)
    dst = edge_index[1].astype(jnp.int32)

    w_ab = jnp.concatenate([msg_w1[:D], msg_w1[D:2 * D]], axis=1)
    w_c = msg_w1[2 * D:]

    a_tab, b_tab = pl.pallas_call(
        _ab_body,
        grid=(10,),
        in_specs=[
            pl.BlockSpec((1000, D), lambda i: (i, 0)),
            pl.BlockSpec((D, 2 * D), lambda i: (0, 0)),
        ],
        out_specs=[
            pl.BlockSpec((1000, D), lambda i: (i, 0)),
            pl.BlockSpec((1000, D), lambda i: (i, 0)),
        ],
        out_shape=[
            jax.ShapeDtypeStruct((N_NODES, D), jnp.float32),
            jax.ShapeDtypeStruct((N_NODES, D), jnp.float32),
        ],
    )(node_feats, w_ab)

    c_tab = pl.pallas_call(
        _c_body,
        grid=(50,),
        in_specs=[
            pl.BlockSpec((6400, D_EDGE), lambda i: (i, 0)),
            pl.BlockSpec((D_EDGE, D), lambda i: (0, 0)),
            pl.BlockSpec((1, D), lambda i: (0, 0)),
        ],
        out_specs=pl.BlockSpec((6400, D), lambda i: (i, 0)),
        out_shape=jax.ShapeDtypeStruct((N_EDGES, D), jnp.float32),
    )(edge_attr, w_c, msg_b1.reshape(1, D))

    sh, deg_tab = _edge_kernel()(a_tab, b_tab, c_tab, src, dst)
    deg = deg_tab.reshape(NC, DROWS * D)[:, :N_NODES].reshape(NC, N_NODES, 1)

    out = pl.pallas_call(
        _post_body,
        grid=(10,),
        in_specs=[
            pl.BlockSpec((NC, 1000, D), lambda i: (0, i, 0)),
            pl.BlockSpec((NC, 1000, 1), lambda i: (0, i, 0)),
            pl.BlockSpec((1000, D), lambda i: (i, 0)),
            pl.BlockSpec((D, D), lambda i: (0, 0)),
            pl.BlockSpec((1, D), lambda i: (0, 0)),
            pl.BlockSpec((D, D), lambda i: (0, 0)),
            pl.BlockSpec((D, D), lambda i: (0, 0)),
            pl.BlockSpec((1, D), lambda i: (0, 0)),
            pl.BlockSpec((D, D), lambda i: (0, 0)),
            pl.BlockSpec((1, D), lambda i: (0, 0)),
        ],
        out_specs=pl.BlockSpec((1000, D), lambda i: (i, 0)),
        out_shape=jax.ShapeDtypeStruct((N_NODES, D), jnp.float32),
    )(sh, deg, node_feats, msg_w2, msg_b2.reshape(1, D),
      upd_w1[:D], upd_w1[D:], upd_b1.reshape(1, D),
      upd_w2, upd_b2.reshape(1, D))

    return out
